# baseline (speedup 1.0000x reference)
import functools

import jax
import jax.numpy as jnp
from jax import lax
from jax.experimental import pallas as pl
from jax.experimental.pallas import tpu as pltpu
from jax.experimental.pallas import tpu_sc as plsc

N = 10000
E = 320000
P = 3
D = 128

NC = 2
NS = 16
NW = NC * NS
EPT = E // NW
CB = 128
NB_E = EPT // CB
TAIL = EPT - NB_E * CB
SEG = 13
NSEG = NB_E // SEG
NPAIR = (SEG - 1) // 2
CZ = 80
NCHUNK = N // CZ
KMAX_S = (NCHUNK + NS - 1) // NS
NTCH = N // CB
KMAX = (NTCH + NW - 1) // NW


def _sc_body(src_hbm, dst_hbm, tsrc_hbm, tdst_hbm, feat_hbm, fidx_hbm,
             z128_hbm, z16_hbm, ones_hbm, acc_out, deg_out, tmp_out,
             acc, dacc, isrc, idst, rows0, rows1, ones_v,
             it_s, it_d, sem0, sem1, sem2, sem3, sem4):
    cid = lax.axis_index("c")
    sid = lax.axis_index("s")
    wid = cid * NS + sid

    pltpu.sync_copy(ones_hbm, ones_v)

    def fire(b, rows, sem):
        return pltpu.async_copy(feat_hbm.at[isrc.at[b]], rows, sem)

    def drain(rows, sem):
        pltpu.make_async_copy(feat_hbm.at[isrc.at[0]], rows, sem).wait()

    def scat(b, rows):
        pltpu.sync_copy(rows, acc.at[idst.at[b]], add=True)
        pltpu.async_copy(ones_v, dacc.at[idst.at[b]], sem2, add=True)

    def scat_async(b, rows, sem):
        pltpu.async_copy(rows, acc.at[idst.at[b]], sem, add=True)
        pltpu.async_copy(ones_v, dacc.at[idst.at[b]], sem2, add=True)

    def wait_scat(rows, sem):
        pltpu.make_async_copy(rows, acc.at[idst.at[0]], sem).wait()

    def drain_ones():
        for _ in range(SEG):
            pltpu.make_async_copy(ones_v, dacc.at[idst.at[0]], sem2).wait()

    for p in range(P):
        for k in range(KMAX_S):
            chunk = sid + NS * k

            @pl.when(chunk < NCHUNK)
            def _():
                off = pl.multiple_of(chunk * CZ, 8)
                pltpu.sync_copy(z128_hbm, acc.at[pl.ds(off, CZ)])
                pltpu.sync_copy(z16_hbm, dacc.at[pl.ds(off, CZ)])

        plsc.subcore_barrier()

        for seg in range(NSEG):
            pltpu.sync_copy(src_hbm.at[p, wid, pl.ds(seg * SEG, SEG)], isrc)
            pltpu.sync_copy(dst_hbm.at[p, wid, pl.ds(seg * SEG, SEG)], idst)
            fire(0, rows0, sem0)
            fire(1, rows1, sem1)

            def body(j, _):
                b = 2 * j
                drain(rows0, sem0)
                scat_async(b, rows0, sem3)
                drain(rows1, sem1)
                scat_async(b + 1, rows1, sem4)
                wait_scat(rows0, sem3)
                fire(b + 2, rows0, sem0)
                wait_scat(rows1, sem4)

                @pl.when(b + 3 < SEG)
                def _():
                    fire(b + 3, rows1, sem1)

                return 0

            lax.fori_loop(0, NPAIR, body, 0)
            drain(rows0, sem0)
            scat(SEG - 1, rows0)
            drain_ones()

        pltpu.sync_copy(tsrc_hbm.at[p, wid], it_s)
        pltpu.sync_copy(tdst_hbm.at[p, wid], it_d)
        pltpu.async_copy(feat_hbm.at[it_s], rows0.at[pl.ds(0, TAIL)],
                         sem0).wait()
        pltpu.sync_copy(rows0.at[pl.ds(0, TAIL)], acc.at[it_d], add=True)
        pltpu.sync_copy(ones_v.at[pl.ds(0, TAIL)], dacc.at[it_d], add=True)
        plsc.subcore_barrier()

        for k in range(KMAX_S):
            chunk = sid + NS * k

            @pl.when(chunk < NCHUNK)
            def _():
                off = pl.multiple_of(chunk * CZ, 8)
                pltpu.sync_copy(acc.at[pl.ds(off, CZ)],
                                acc_out.at[p, cid, pl.ds(off, CZ)])
                pltpu.sync_copy(dacc.at[pl.ds(off, CZ)],
                                deg_out.at[p, cid, pl.ds(off, CZ)])

        plsc.subcore_barrier()

    for k in range(KMAX):
        chunk = wid + NW * k

        @pl.when(chunk < NTCH)
        def _():
            off = pl.multiple_of(chunk * CB, 8)
            pltpu.sync_copy(fidx_hbm.at[pl.ds(off, CB)], isrc.at[0])
            pltpu.async_copy(feat_hbm.at[isrc.at[0]], rows0, sem0).wait()
            pltpu.sync_copy(rows0, tmp_out.at[pl.ds(off, CB)])

    @pl.when(wid == 0)
    def _():
        off = pl.multiple_of(NTCH * CB, 8)
        pltpu.sync_copy(fidx_hbm.at[pl.ds(off, TAIL)], it_s)
        pltpu.async_copy(feat_hbm.at[it_s], rows0.at[pl.ds(0, TAIL)],
                         sem0).wait()
        pltpu.sync_copy(rows0.at[pl.ds(0, TAIL)], tmp_out.at[pl.ds(off, TAIL)])


_sc_call = functools.partial(
    pl.kernel,
    mesh=plsc.VectorSubcoreMesh(core_axis_name="c", subcore_axis_name="s"),
    out_type=[
        jax.ShapeDtypeStruct((P, NC, N, D), jnp.float32),
        jax.ShapeDtypeStruct((P, NC, N, 16), jnp.float32),
        jax.ShapeDtypeStruct((N, D), jnp.float32),
    ],
    scratch_types=[
        pltpu.VMEM_SHARED((N, D), jnp.float32),
        pltpu.VMEM_SHARED((N, 16), jnp.float32),
        pltpu.VMEM((SEG, CB), jnp.int32),
        pltpu.VMEM((SEG, CB), jnp.int32),
        pltpu.VMEM((CB, D), jnp.float32),
        pltpu.VMEM((CB, D), jnp.float32),
        pltpu.VMEM((CB, 16), jnp.float32),
        pltpu.VMEM((TAIL,), jnp.int32),
        pltpu.VMEM((TAIL,), jnp.int32),
        pltpu.SemaphoreType.DMA,
        pltpu.SemaphoreType.DMA,
        pltpu.SemaphoreType.DMA,
        pltpu.SemaphoreType.DMA,
        pltpu.SemaphoreType.DMA,
    ],
    compiler_params=pltpu.CompilerParams(use_tc_tiling_on_sc=False),
)(_sc_body)


R = 1000
NBLK = N // R


def _tc_reduce_body(acc_ref, deg_ref, m0_ref, agg_ref, stats_ref):
    i = pl.program_id(0)
    a = acc_ref[...]
    d = deg_ref[...]
    deg = d[:, 0, :, 0:1] + d[:, 1, :, 0:1]
    agg = (a[:, 0] + a[:, 1]) / jnp.maximum(deg, 1.0)
    agg_ref[...] = agg
    m = m0_ref[...]
    contrib = jnp.sum(jnp.tanh(agg) * m[None, :, :], axis=1)
    cnt = jnp.sum(m)

    @pl.when(i == 0)
    def _():
        stats_ref[...] = jnp.zeros((8, D), jnp.float32)

    upd = jnp.concatenate(
        [contrib, jnp.full((1, D), cnt, jnp.float32),
         jnp.zeros((4, D), jnp.float32)], axis=0)
    stats_ref[...] = stats_ref[...] + upd


def _tc_final_body(agg_ref, stats_ref, att_ref, m0_ref, tmp_ref,
                   fcwt_ref, fcb_ref, hfc_ref, h_ref):
    stats = stats_ref[...]
    att = att_ref[...]
    sv = jnp.sum(stats[0:P, :] * att, axis=1, keepdims=True)
    cnt = jnp.maximum(stats[P:P + 1, 0:1], 1.0)
    s = sv / cnt
    s = s - jnp.max(s, axis=0, keepdims=True)
    e = jnp.exp(s)
    beta = e / jnp.sum(e, axis=0, keepdims=True)
    agg = agg_ref[...]
    ht = (beta[0:1, 0:1] * agg[0] + beta[1:2, 0:1] * agg[1]
          + beta[2:3, 0:1] * agg[2])
    h = jnp.where(m0_ref[...] > 0.5, ht, tmp_ref[...])
    h_ref[...] = h
    hfc_ref[...] = (jnp.dot(h, fcwt_ref[...],
                            preferred_element_type=jnp.float32)
                    + fcb_ref[...])


def kernel(features, type_mask, adj_matrixes, feature_idxes, fc_w, fc_b, att):
    f32 = jnp.float32
    m0 = (type_mask == 0).astype(f32).reshape(N, 1)
    z128 = jnp.zeros((CZ, D), f32)
    z16 = jnp.zeros((CZ, 16), f32)
    ones16 = jnp.ones((CB, 16), f32)
    src3 = adj_matrixes[:, 0, :].reshape(P, NW, EPT)
    dst3 = adj_matrixes[:, 1, :].reshape(P, NW, EPT)
    src4 = src3[:, :, :NB_E * CB].reshape(P, NW, NB_E, CB)
    dst4 = dst3[:, :, :NB_E * CB].reshape(P, NW, NB_E, CB)
    tsrc = src3[:, :, NB_E * CB:]
    tdst = dst3[:, :, NB_E * CB:]

    acc, deg, temps = _sc_call(src4, dst4, tsrc, tdst, features,
                               feature_idxes, z128, z16, ones16)

    agg, stats = pl.pallas_call(
        _tc_reduce_body,
        grid=(NBLK,),
        in_specs=[
            pl.BlockSpec((P, NC, R, D), lambda i: (0, 0, i, 0)),
            pl.BlockSpec((P, NC, R, 16), lambda i: (0, 0, i, 0)),
            pl.BlockSpec((R, 1), lambda i: (i, 0)),
        ],
        out_specs=[
            pl.BlockSpec((P, R, D), lambda i: (0, i, 0)),
            pl.BlockSpec((8, D), lambda i: (0, 0)),
        ],
        out_shape=[
            jax.ShapeDtypeStruct((P, N, D), f32),
            jax.ShapeDtypeStruct((8, D), f32),
        ],
    )(acc, deg, m0)

    hfc, h = pl.pallas_call(
        _tc_final_body,
        grid=(NBLK,),
        in_specs=[
            pl.BlockSpec((P, R, D), lambda i: (0, i, 0)),
            pl.BlockSpec((8, D), lambda i: (0, 0)),
            pl.BlockSpec((P, D), lambda i: (0, 0)),
            pl.BlockSpec((R, 1), lambda i: (i, 0)),
            pl.BlockSpec((R, D), lambda i: (i, 0)),
            pl.BlockSpec((D, D), lambda i: (0, 0)),
            pl.BlockSpec((1, D), lambda i: (0, 0)),
        ],
        out_specs=[
            pl.BlockSpec((R, D), lambda i: (i, 0)),
            pl.BlockSpec((R, D), lambda i: (i, 0)),
        ],
        out_shape=[
            jax.ShapeDtypeStruct((N, D), f32),
            jax.ShapeDtypeStruct((N, D), f32),
        ],
    )(agg, stats, att, m0, temps, fc_w.T, fc_b.reshape(1, D))

    return (hfc, h)


# speedup vs baseline: 1.0736x; 1.0736x over previous
import functools

import jax
import jax.numpy as jnp
from jax import lax
from jax.experimental import pallas as pl
from jax.experimental.pallas import tpu as pltpu
from jax.experimental.pallas import tpu_sc as plsc

N = 10000
E = 320000
P = 3
D = 128

NC = 2
NS = 16
NW = NC * NS
EPT = E // NW
CB = 128
NB_E = EPT // CB
TAIL = EPT - NB_E * CB
SEG = 13
NSEG = NB_E // SEG
NPAIR = (SEG - 1) // 2
CZ = 80
NCHUNK = N // CZ
KMAX_S = (NCHUNK + NS - 1) // NS
NTCH = N // CB
KMAX = (NTCH + NW - 1) // NW


def _sc_body(src_hbm, dst_hbm, tsrc_hbm, tdst_hbm, feat_hbm, fidx_hbm,
             z128_hbm, z16_hbm, ones_hbm, acc_out, deg_out, tmp_out,
             acc, dacc, isrc, idst, rows0, rows1, ones_v,
             it_s, it_d, sem0, sem1, sem2):
    cid = lax.axis_index("c")
    sid = lax.axis_index("s")
    wid = cid * NS + sid

    pltpu.sync_copy(ones_hbm, ones_v)

    def fire(b, rows, sem):
        return pltpu.async_copy(feat_hbm.at[isrc.at[b]], rows, sem)

    def drain(rows, sem):
        pltpu.make_async_copy(feat_hbm.at[isrc.at[0]], rows, sem).wait()

    def scat(b, rows):
        pltpu.sync_copy(rows, acc.at[idst.at[b]], add=True)
        pltpu.async_copy(ones_v, dacc.at[idst.at[b]], sem2, add=True)

    def drain_ones():
        for _ in range(SEG):
            pltpu.make_async_copy(ones_v, dacc.at[idst.at[0]], sem2).wait()

    for p in range(P):
        for k in range(KMAX_S):
            chunk = sid + NS * k

            @pl.when(chunk < NCHUNK)
            def _():
                off = pl.multiple_of(chunk * CZ, 8)
                pltpu.sync_copy(z128_hbm, acc.at[pl.ds(off, CZ)])
                pltpu.sync_copy(z16_hbm, dacc.at[pl.ds(off, CZ)])

        plsc.subcore_barrier()

        for seg in range(NSEG):
            pltpu.sync_copy(src_hbm.at[p, wid, pl.ds(seg * SEG, SEG)], isrc)
            pltpu.sync_copy(dst_hbm.at[p, wid, pl.ds(seg * SEG, SEG)], idst)
            fire(0, rows0, sem0)

            def body(j, _):
                b = 2 * j
                fire(b + 1, rows1, sem1)
                drain(rows0, sem0)
                scat(b, rows0)
                fire(b + 2, rows0, sem0)
                drain(rows1, sem1)
                scat(b + 1, rows1)
                return 0

            lax.fori_loop(0, NPAIR, body, 0)
            drain(rows0, sem0)
            scat(SEG - 1, rows0)
            drain_ones()

        pltpu.sync_copy(tsrc_hbm.at[p, wid], it_s)
        pltpu.sync_copy(tdst_hbm.at[p, wid], it_d)
        pltpu.async_copy(feat_hbm.at[it_s], rows0.at[pl.ds(0, TAIL)],
                         sem0).wait()
        pltpu.sync_copy(rows0.at[pl.ds(0, TAIL)], acc.at[it_d], add=True)
        pltpu.sync_copy(ones_v.at[pl.ds(0, TAIL)], dacc.at[it_d], add=True)
        plsc.subcore_barrier()

        for k in range(KMAX_S):
            chunk = sid + NS * k

            @pl.when(chunk < NCHUNK)
            def _():
                off = pl.multiple_of(chunk * CZ, 8)
                pltpu.sync_copy(acc.at[pl.ds(off, CZ)],
                                acc_out.at[p, cid, pl.ds(off, CZ)])
                pltpu.sync_copy(dacc.at[pl.ds(off, CZ)],
                                deg_out.at[p, cid, pl.ds(off, CZ)])

        plsc.subcore_barrier()

    for k in range(KMAX):
        chunk = wid + NW * k

        @pl.when(chunk < NTCH)
        def _():
            off = pl.multiple_of(chunk * CB, 8)
            pltpu.sync_copy(fidx_hbm.at[pl.ds(off, CB)], isrc.at[0])
            pltpu.async_copy(feat_hbm.at[isrc.at[0]], rows0, sem0).wait()
            pltpu.sync_copy(rows0, tmp_out.at[pl.ds(off, CB)])

    @pl.when(wid == 0)
    def _():
        off = pl.multiple_of(NTCH * CB, 8)
        pltpu.sync_copy(fidx_hbm.at[pl.ds(off, TAIL)], it_s)
        pltpu.async_copy(feat_hbm.at[it_s], rows0.at[pl.ds(0, TAIL)],
                         sem0).wait()
        pltpu.sync_copy(rows0.at[pl.ds(0, TAIL)], tmp_out.at[pl.ds(off, TAIL)])


_sc_call = functools.partial(
    pl.kernel,
    mesh=plsc.VectorSubcoreMesh(core_axis_name="c", subcore_axis_name="s"),
    out_type=[
        jax.ShapeDtypeStruct((P, NC, N, D), jnp.float32),
        jax.ShapeDtypeStruct((P, NC, N, 16), jnp.float32),
        jax.ShapeDtypeStruct((N, D), jnp.float32),
    ],
    scratch_types=[
        pltpu.VMEM_SHARED((N, D), jnp.float32),
        pltpu.VMEM_SHARED((N, 16), jnp.float32),
        pltpu.VMEM((SEG, CB), jnp.int32),
        pltpu.VMEM((SEG, CB), jnp.int32),
        pltpu.VMEM((CB, D), jnp.float32),
        pltpu.VMEM((CB, D), jnp.float32),
        pltpu.VMEM((CB, 16), jnp.float32),
        pltpu.VMEM((TAIL,), jnp.int32),
        pltpu.VMEM((TAIL,), jnp.int32),
        pltpu.SemaphoreType.DMA,
        pltpu.SemaphoreType.DMA,
        pltpu.SemaphoreType.DMA,
    ],
    compiler_params=pltpu.CompilerParams(use_tc_tiling_on_sc=False),
)(_sc_body)


R = 1000
NBLK = N // R


def _tc_reduce_body(acc_ref, deg_ref, m0_ref, agg_ref, stats_ref):
    i = pl.program_id(0)
    a = acc_ref[...]
    d = deg_ref[...]
    deg = d[:, 0, :, 0:1] + d[:, 1, :, 0:1]
    agg = (a[:, 0] + a[:, 1]) / jnp.maximum(deg, 1.0)
    agg_ref[...] = agg
    m = m0_ref[...]
    contrib = jnp.sum(jnp.tanh(agg) * m[None, :, :], axis=1)
    cnt = jnp.sum(m)

    @pl.when(i == 0)
    def _():
        stats_ref[...] = jnp.zeros((8, D), jnp.float32)

    upd = jnp.concatenate(
        [contrib, jnp.full((1, D), cnt, jnp.float32),
         jnp.zeros((4, D), jnp.float32)], axis=0)
    stats_ref[...] = stats_ref[...] + upd


def _tc_final_body(agg_ref, stats_ref, att_ref, m0_ref, tmp_ref,
                   fcwt_ref, fcb_ref, hfc_ref, h_ref):
    stats = stats_ref[...]
    att = att_ref[...]
    sv = jnp.sum(stats[0:P, :] * att, axis=1, keepdims=True)
    cnt = jnp.maximum(stats[P:P + 1, 0:1], 1.0)
    s = sv / cnt
    s = s - jnp.max(s, axis=0, keepdims=True)
    e = jnp.exp(s)
    beta = e / jnp.sum(e, axis=0, keepdims=True)
    agg = agg_ref[...]
    ht = (beta[0:1, 0:1] * agg[0] + beta[1:2, 0:1] * agg[1]
          + beta[2:3, 0:1] * agg[2])
    h = jnp.where(m0_ref[...] > 0.5, ht, tmp_ref[...])
    h_ref[...] = h
    hfc_ref[...] = (jnp.dot(h, fcwt_ref[...],
                            preferred_element_type=jnp.float32)
                    + fcb_ref[...])


def kernel(features, type_mask, adj_matrixes, feature_idxes, fc_w, fc_b, att):
    f32 = jnp.float32
    m0 = (type_mask == 0).astype(f32).reshape(N, 1)
    z128 = jnp.zeros((CZ, D), f32)
    z16 = jnp.zeros((CZ, 16), f32)
    ones16 = jnp.ones((CB, 16), f32)
    src3 = adj_matrixes[:, 0, :].reshape(P, NW, EPT)
    dst3 = adj_matrixes[:, 1, :].reshape(P, NW, EPT)
    src4 = src3[:, :, :NB_E * CB].reshape(P, NW, NB_E, CB)
    dst4 = dst3[:, :, :NB_E * CB].reshape(P, NW, NB_E, CB)
    tsrc = src3[:, :, NB_E * CB:]
    tdst = dst3[:, :, NB_E * CB:]

    acc, deg, temps = _sc_call(src4, dst4, tsrc, tdst, features,
                               feature_idxes, z128, z16, ones16)

    agg, stats = pl.pallas_call(
        _tc_reduce_body,
        grid=(NBLK,),
        in_specs=[
            pl.BlockSpec((P, NC, R, D), lambda i: (0, 0, i, 0)),
            pl.BlockSpec((P, NC, R, 16), lambda i: (0, 0, i, 0)),
            pl.BlockSpec((R, 1), lambda i: (i, 0)),
        ],
        out_specs=[
            pl.BlockSpec((P, R, D), lambda i: (0, i, 0)),
            pl.BlockSpec((8, D), lambda i: (0, 0)),
        ],
        out_shape=[
            jax.ShapeDtypeStruct((P, N, D), f32),
            jax.ShapeDtypeStruct((8, D), f32),
        ],
    )(acc, deg, m0)

    hfc, h = pl.pallas_call(
        _tc_final_body,
        grid=(NBLK,),
        in_specs=[
            pl.BlockSpec((P, R, D), lambda i: (0, i, 0)),
            pl.BlockSpec((8, D), lambda i: (0, 0)),
            pl.BlockSpec((P, D), lambda i: (0, 0)),
            pl.BlockSpec((R, 1), lambda i: (i, 0)),
            pl.BlockSpec((R, D), lambda i: (i, 0)),
            pl.BlockSpec((D, D), lambda i: (0, 0)),
            pl.BlockSpec((1, D), lambda i: (0, 0)),
        ],
        out_specs=[
            pl.BlockSpec((R, D), lambda i: (i, 0)),
            pl.BlockSpec((R, D), lambda i: (i, 0)),
        ],
        out_shape=[
            jax.ShapeDtypeStruct((N, D), f32),
            jax.ShapeDtypeStruct((N, D), f32),
        ],
    )(agg, stats, att, m0, temps, fc_w.T, fc_b.reshape(1, D))

    return (hfc, h)


# speedup vs baseline: 1.1566x; 1.0773x over previous
import functools

import jax
import jax.numpy as jnp
from jax import lax
from jax.experimental import pallas as pl
from jax.experimental.pallas import tpu as pltpu
from jax.experimental.pallas import tpu_sc as plsc

N = 10000
E = 320000
P = 3
D = 128

NC = 2
NS = 16
NW = NC * NS
EPT = E // NW
CB = 128
NB_E = EPT // CB
TAIL = EPT - NB_E * CB
SEG = 13
NSEG = NB_E // SEG
NPAIR = (SEG - 1) // 2
CZ = 80
NCHUNK = N // CZ
KMAX_S = (NCHUNK + NS - 1) // NS
NTCH = N // CB
KMAX = (NTCH + NW - 1) // NW


def _sc_body(src_hbm, dst_hbm, tsrc_hbm, tdst_hbm, feat_hbm, fidx_hbm,
             z128_hbm, z16_hbm, ones_hbm, acc_out, deg_out, tmp_out,
             acc, dacc, isrc, idst, rows0, rows1, ones_v, buf16,
             it_s, it_d, sem0, sem1, sem2):
    cid = lax.axis_index("c")
    sid = lax.axis_index("s")
    wid = cid * NS + sid

    pltpu.sync_copy(ones_hbm, ones_v)

    def fire(b, rows, sem):
        return pltpu.async_copy(feat_hbm.at[isrc.at[b]], rows, sem)

    def drain(rows, sem):
        pltpu.make_async_copy(feat_hbm.at[isrc.at[0]], rows, sem).wait()

    def scat(b, rows):
        pltpu.sync_copy(rows, acc.at[idst.at[b]], add=True)
        pltpu.async_copy(ones_v, dacc.at[idst.at[b]], sem2, add=True)

    def drain_ones():
        for _ in range(SEG):
            pltpu.make_async_copy(ones_v, dacc.at[idst.at[0]], sem2).wait()

    for p in range(P):
        pltpu.sync_copy(z128_hbm, rows0.at[pl.ds(0, CZ)])
        pltpu.sync_copy(z16_hbm, buf16)
        for k in range(KMAX_S):
            chunk = sid + NS * k

            @pl.when(chunk < NCHUNK)
            def _():
                off = pl.multiple_of(chunk * CZ, 8)
                pltpu.sync_copy(rows0.at[pl.ds(0, CZ)], acc.at[pl.ds(off, CZ)])
                pltpu.sync_copy(buf16, dacc.at[pl.ds(off, CZ)])

        plsc.subcore_barrier()

        for seg in range(NSEG):
            pltpu.sync_copy(src_hbm.at[p, wid, pl.ds(seg * SEG, SEG)], isrc)
            pltpu.sync_copy(dst_hbm.at[p, wid, pl.ds(seg * SEG, SEG)], idst)
            fire(0, rows0, sem0)

            def body(j, _):
                b = 2 * j
                fire(b + 1, rows1, sem1)
                drain(rows0, sem0)
                scat(b, rows0)
                fire(b + 2, rows0, sem0)
                drain(rows1, sem1)
                scat(b + 1, rows1)
                return 0

            lax.fori_loop(0, NPAIR, body, 0)
            drain(rows0, sem0)
            scat(SEG - 1, rows0)
            drain_ones()

        pltpu.sync_copy(tsrc_hbm.at[p, wid], it_s)
        pltpu.sync_copy(tdst_hbm.at[p, wid], it_d)
        pltpu.async_copy(feat_hbm.at[it_s], rows0.at[pl.ds(0, TAIL)],
                         sem0).wait()
        pltpu.sync_copy(rows0.at[pl.ds(0, TAIL)], acc.at[it_d], add=True)
        pltpu.sync_copy(ones_v.at[pl.ds(0, TAIL)], dacc.at[it_d], add=True)
        plsc.subcore_barrier()

        for k in range(KMAX_S):
            chunk = sid + NS * k

            @pl.when(chunk < NCHUNK)
            def _():
                off = pl.multiple_of(chunk * CZ, 8)
                pltpu.sync_copy(acc.at[pl.ds(off, CZ)],
                                acc_out.at[p, cid, pl.ds(off, CZ)])
                pltpu.sync_copy(dacc.at[pl.ds(off, CZ)],
                                deg_out.at[p, cid, pl.ds(off, CZ)])

        plsc.subcore_barrier()

    for k in range(KMAX):
        chunk = wid + NW * k

        @pl.when(chunk < NTCH)
        def _():
            off = pl.multiple_of(chunk * CB, 8)
            pltpu.sync_copy(fidx_hbm.at[pl.ds(off, CB)], isrc.at[0])
            pltpu.async_copy(feat_hbm.at[isrc.at[0]], rows0, sem0).wait()
            pltpu.sync_copy(rows0, tmp_out.at[pl.ds(off, CB)])

    @pl.when(wid == 0)
    def _():
        off = pl.multiple_of(NTCH * CB, 8)
        pltpu.sync_copy(fidx_hbm.at[pl.ds(off, TAIL)], it_s)
        pltpu.async_copy(feat_hbm.at[it_s], rows0.at[pl.ds(0, TAIL)],
                         sem0).wait()
        pltpu.sync_copy(rows0.at[pl.ds(0, TAIL)], tmp_out.at[pl.ds(off, TAIL)])


_sc_call = functools.partial(
    pl.kernel,
    mesh=plsc.VectorSubcoreMesh(core_axis_name="c", subcore_axis_name="s"),
    out_type=[
        jax.ShapeDtypeStruct((P, NC, N, D), jnp.float32),
        jax.ShapeDtypeStruct((P, NC, N, 16), jnp.float32),
        jax.ShapeDtypeStruct((N, D), jnp.float32),
    ],
    scratch_types=[
        pltpu.VMEM_SHARED((N, D), jnp.float32),
        pltpu.VMEM_SHARED((N, 16), jnp.float32),
        pltpu.VMEM((SEG, CB), jnp.int32),
        pltpu.VMEM((SEG, CB), jnp.int32),
        pltpu.VMEM((CB, D), jnp.float32),
        pltpu.VMEM((CB, D), jnp.float32),
        pltpu.VMEM((CB, 16), jnp.float32),
        pltpu.VMEM((CZ, 16), jnp.float32),
        pltpu.VMEM((TAIL,), jnp.int32),
        pltpu.VMEM((TAIL,), jnp.int32),
        pltpu.SemaphoreType.DMA,
        pltpu.SemaphoreType.DMA,
        pltpu.SemaphoreType.DMA,
    ],
    compiler_params=pltpu.CompilerParams(use_tc_tiling_on_sc=False),
)(_sc_body)


R = 1000
NBLK = N // R


def _tc_reduce_body(acc_ref, deg_ref, m0_ref, agg_ref, stats_ref):
    i = pl.program_id(0)
    a = acc_ref[...]
    d = deg_ref[...]
    deg = d[:, 0, :, 0:1] + d[:, 1, :, 0:1]
    agg = (a[:, 0] + a[:, 1]) / jnp.maximum(deg, 1.0)
    agg_ref[...] = agg
    m = m0_ref[...]
    contrib = jnp.sum(jnp.tanh(agg) * m[None, :, :], axis=1)
    cnt = jnp.sum(m)

    @pl.when(i == 0)
    def _():
        stats_ref[...] = jnp.zeros((8, D), jnp.float32)

    upd = jnp.concatenate(
        [contrib, jnp.full((1, D), cnt, jnp.float32),
         jnp.zeros((4, D), jnp.float32)], axis=0)
    stats_ref[...] = stats_ref[...] + upd


def _tc_final_body(agg_ref, stats_ref, att_ref, m0_ref, tmp_ref,
                   fcwt_ref, fcb_ref, hfc_ref, h_ref):
    stats = stats_ref[...]
    att = att_ref[...]
    sv = jnp.sum(stats[0:P, :] * att, axis=1, keepdims=True)
    cnt = jnp.maximum(stats[P:P + 1, 0:1], 1.0)
    s = sv / cnt
    s = s - jnp.max(s, axis=0, keepdims=True)
    e = jnp.exp(s)
    beta = e / jnp.sum(e, axis=0, keepdims=True)
    agg = agg_ref[...]
    ht = (beta[0:1, 0:1] * agg[0] + beta[1:2, 0:1] * agg[1]
          + beta[2:3, 0:1] * agg[2])
    h = jnp.where(m0_ref[...] > 0.5, ht, tmp_ref[...])
    h_ref[...] = h
    hfc_ref[...] = (jnp.dot(h, fcwt_ref[...],
                            preferred_element_type=jnp.float32)
                    + fcb_ref[...])


def kernel(features, type_mask, adj_matrixes, feature_idxes, fc_w, fc_b, att):
    f32 = jnp.float32
    m0 = (type_mask == 0).astype(f32).reshape(N, 1)
    z128 = jnp.zeros((CZ, D), f32)
    z16 = jnp.zeros((CZ, 16), f32)
    ones16 = jnp.ones((CB, 16), f32)
    src3 = adj_matrixes[:, 0, :].reshape(P, NW, EPT)
    dst3 = adj_matrixes[:, 1, :].reshape(P, NW, EPT)
    src4 = src3[:, :, :NB_E * CB].reshape(P, NW, NB_E, CB)
    dst4 = dst3[:, :, :NB_E * CB].reshape(P, NW, NB_E, CB)
    tsrc = src3[:, :, NB_E * CB:]
    tdst = dst3[:, :, NB_E * CB:]

    acc, deg, temps = _sc_call(src4, dst4, tsrc, tdst, features,
                               feature_idxes, z128, z16, ones16)

    agg, stats = pl.pallas_call(
        _tc_reduce_body,
        grid=(NBLK,),
        in_specs=[
            pl.BlockSpec((P, NC, R, D), lambda i: (0, 0, i, 0)),
            pl.BlockSpec((P, NC, R, 16), lambda i: (0, 0, i, 0)),
            pl.BlockSpec((R, 1), lambda i: (i, 0)),
        ],
        out_specs=[
            pl.BlockSpec((P, R, D), lambda i: (0, i, 0)),
            pl.BlockSpec((8, D), lambda i: (0, 0)),
        ],
        out_shape=[
            jax.ShapeDtypeStruct((P, N, D), f32),
            jax.ShapeDtypeStruct((8, D), f32),
        ],
    )(acc, deg, m0)

    hfc, h = pl.pallas_call(
        _tc_final_body,
        grid=(NBLK,),
        in_specs=[
            pl.BlockSpec((P, R, D), lambda i: (0, i, 0)),
            pl.BlockSpec((8, D), lambda i: (0, 0)),
            pl.BlockSpec((P, D), lambda i: (0, 0)),
            pl.BlockSpec((R, 1), lambda i: (i, 0)),
            pl.BlockSpec((R, D), lambda i: (i, 0)),
            pl.BlockSpec((D, D), lambda i: (0, 0)),
            pl.BlockSpec((1, D), lambda i: (0, 0)),
        ],
        out_specs=[
            pl.BlockSpec((R, D), lambda i: (i, 0)),
            pl.BlockSpec((R, D), lambda i: (i, 0)),
        ],
        out_shape=[
            jax.ShapeDtypeStruct((N, D), f32),
            jax.ShapeDtypeStruct((N, D), f32),
        ],
    )(agg, stats, att, m0, temps, fc_w.T, fc_b.reshape(1, D))

    return (hfc, h)


# speedup vs baseline: 1.2846x; 1.1106x over previous
import functools

import jax
import jax.numpy as jnp
from jax import lax
from jax.experimental import pallas as pl
from jax.experimental.pallas import tpu as pltpu
from jax.experimental.pallas import tpu_sc as plsc

N = 10000
E = 320000
P = 3
D = 128

NC = 2
NS = 16
NW = NC * NS
EPT = E // NW
CB = 80
NB_E = EPT // CB
SEG = 25
NSEG = NB_E // SEG
NTRI = (SEG - 4) // 3
CZ = 80
NCHUNK = N // CZ
KMAX_S = (NCHUNK + NS - 1) // NS
KMAX = (NCHUNK + NW - 1) // NW


def _sc_body(src_hbm, dst_hbm, feat_hbm, fidx_hbm, z128_hbm, z16_hbm,
             ones_hbm, acc_out, deg_out, tmp_out,
             acc, dacc, isrc, idst, rows0, rows1, rows2, ones_v, buf16,
             semA, semB, semC, semO):
    cid = lax.axis_index("c")
    sid = lax.axis_index("s")
    wid = cid * NS + sid

    pltpu.sync_copy(ones_hbm, ones_v)

    def fire(b, rows, sem):
        return pltpu.async_copy(feat_hbm.at[isrc.at[b]], rows, sem)

    def drain(rows, sem):
        pltpu.make_async_copy(feat_hbm.at[isrc.at[0]], rows, sem).wait()

    def scat(b, rows):
        pltpu.sync_copy(rows, acc.at[idst.at[b]], add=True)
        pltpu.async_copy(ones_v, dacc.at[idst.at[b]], semO, add=True)

    def drain_ones():
        for _ in range(SEG):
            pltpu.make_async_copy(ones_v, dacc.at[idst.at[0]], semO).wait()

    def zero_chunks():
        pltpu.sync_copy(z128_hbm, rows0)
        pltpu.sync_copy(z16_hbm, buf16)
        for k in range(KMAX_S):
            chunk = sid + NS * k

            @pl.when(chunk < NCHUNK)
            def _():
                off = pl.multiple_of(chunk * CZ, 8)
                pltpu.sync_copy(rows0, acc.at[pl.ds(off, CZ)])
                pltpu.sync_copy(buf16, dacc.at[pl.ds(off, CZ)])

    zero_chunks()
    plsc.subcore_barrier()

    for p in range(P):
        for seg in range(NSEG):
            pltpu.sync_copy(src_hbm.at[p, wid, pl.ds(seg * SEG, SEG)], isrc)
            pltpu.sync_copy(dst_hbm.at[p, wid, pl.ds(seg * SEG, SEG)], idst)
            fire(0, rows0, semA)
            fire(1, rows1, semB)
            fire(2, rows2, semC)

            def body(j, _):
                b = 3 * j
                drain(rows0, semA)
                scat(b, rows0)
                fire(b + 3, rows0, semA)
                drain(rows1, semB)
                scat(b + 1, rows1)
                fire(b + 4, rows1, semB)
                drain(rows2, semC)
                scat(b + 2, rows2)
                fire(b + 5, rows2, semC)
                return 0

            lax.fori_loop(0, NTRI, body, 0)
            bq = 3 * NTRI
            drain(rows0, semA)
            scat(bq, rows0)
            fire(bq + 3, rows0, semA)
            drain(rows1, semB)
            scat(bq + 1, rows1)
            drain(rows2, semC)
            scat(bq + 2, rows2)
            drain(rows0, semA)
            scat(bq + 3, rows0)
            drain_ones()

        plsc.subcore_barrier()

        for k in range(KMAX_S):
            chunk = sid + NS * k

            @pl.when(chunk < NCHUNK)
            def _():
                off = pl.multiple_of(chunk * CZ, 8)
                pltpu.sync_copy(acc.at[pl.ds(off, CZ)],
                                acc_out.at[p, cid, pl.ds(off, CZ)])
                pltpu.sync_copy(dacc.at[pl.ds(off, CZ)],
                                deg_out.at[p, cid, pl.ds(off, CZ)])

        if p < P - 1:
            zero_chunks()
        plsc.subcore_barrier()

    for k in range(KMAX):
        chunk = wid + NW * k

        @pl.when(chunk < NCHUNK)
        def _():
            off = pl.multiple_of(chunk * CB, 8)
            pltpu.sync_copy(fidx_hbm.at[pl.ds(off, CB)], isrc.at[0])
            pltpu.async_copy(feat_hbm.at[isrc.at[0]], rows0, semA).wait()
            pltpu.sync_copy(rows0, tmp_out.at[pl.ds(off, CB)])


_sc_call = functools.partial(
    pl.kernel,
    mesh=plsc.VectorSubcoreMesh(core_axis_name="c", subcore_axis_name="s"),
    out_type=[
        jax.ShapeDtypeStruct((P, NC, N, D), jnp.float32),
        jax.ShapeDtypeStruct((P, NC, N, 16), jnp.float32),
        jax.ShapeDtypeStruct((N, D), jnp.float32),
    ],
    scratch_types=[
        pltpu.VMEM_SHARED((N, D), jnp.float32),
        pltpu.VMEM_SHARED((N, 16), jnp.float32),
        pltpu.VMEM((SEG, CB), jnp.int32),
        pltpu.VMEM((SEG, CB), jnp.int32),
        pltpu.VMEM((CB, D), jnp.float32),
        pltpu.VMEM((CB, D), jnp.float32),
        pltpu.VMEM((CB, D), jnp.float32),
        pltpu.VMEM((CB, 16), jnp.float32),
        pltpu.VMEM((CZ, 16), jnp.float32),
        pltpu.SemaphoreType.DMA,
        pltpu.SemaphoreType.DMA,
        pltpu.SemaphoreType.DMA,
        pltpu.SemaphoreType.DMA,
    ],
    compiler_params=pltpu.CompilerParams(use_tc_tiling_on_sc=False),
)(_sc_body)


R = 1000
NBLK = N // R


def _tc_reduce_body(acc_ref, deg_ref, m0_ref, agg_ref, stats_ref):
    i = pl.program_id(0)
    a = acc_ref[...]
    d = deg_ref[...]
    deg = d[:, 0, :, 0:1] + d[:, 1, :, 0:1]
    agg = (a[:, 0] + a[:, 1]) / jnp.maximum(deg, 1.0)
    agg_ref[...] = agg
    m = m0_ref[...]
    contrib = jnp.sum(jnp.tanh(agg) * m[None, :, :], axis=1)
    cnt = jnp.sum(m)

    @pl.when(i == 0)
    def _():
        stats_ref[...] = jnp.zeros((8, D), jnp.float32)

    upd = jnp.concatenate(
        [contrib, jnp.full((1, D), cnt, jnp.float32),
         jnp.zeros((4, D), jnp.float32)], axis=0)
    stats_ref[...] = stats_ref[...] + upd


def _tc_final_body(agg_ref, stats_ref, att_ref, m0_ref, tmp_ref,
                   fcwt_ref, fcb_ref, hfc_ref, h_ref):
    stats = stats_ref[...]
    att = att_ref[...]
    sv = jnp.sum(stats[0:P, :] * att, axis=1, keepdims=True)
    cnt = jnp.maximum(stats[P:P + 1, 0:1], 1.0)
    s = sv / cnt
    s = s - jnp.max(s, axis=0, keepdims=True)
    e = jnp.exp(s)
    beta = e / jnp.sum(e, axis=0, keepdims=True)
    agg = agg_ref[...]
    ht = (beta[0:1, 0:1] * agg[0] + beta[1:2, 0:1] * agg[1]
          + beta[2:3, 0:1] * agg[2])
    h = jnp.where(m0_ref[...] > 0.5, ht, tmp_ref[...])
    h_ref[...] = h
    hfc_ref[...] = (jnp.dot(h, fcwt_ref[...],
                            preferred_element_type=jnp.float32)
                    + fcb_ref[...])


def kernel(features, type_mask, adj_matrixes, feature_idxes, fc_w, fc_b, att):
    f32 = jnp.float32
    m0 = (type_mask == 0).astype(f32).reshape(N, 1)
    z128 = jnp.zeros((CZ, D), f32)
    z16 = jnp.zeros((CZ, 16), f32)
    ones16 = jnp.ones((CB, 16), f32)
    src4 = adj_matrixes[:, 0, :].reshape(P, NW, NB_E, CB)
    dst4 = adj_matrixes[:, 1, :].reshape(P, NW, NB_E, CB)

    acc, deg, temps = _sc_call(src4, dst4, features, feature_idxes,
                               z128, z16, ones16)

    agg, stats = pl.pallas_call(
        _tc_reduce_body,
        grid=(NBLK,),
        in_specs=[
            pl.BlockSpec((P, NC, R, D), lambda i: (0, 0, i, 0)),
            pl.BlockSpec((P, NC, R, 16), lambda i: (0, 0, i, 0)),
            pl.BlockSpec((R, 1), lambda i: (i, 0)),
        ],
        out_specs=[
            pl.BlockSpec((P, R, D), lambda i: (0, i, 0)),
            pl.BlockSpec((8, D), lambda i: (0, 0)),
        ],
        out_shape=[
            jax.ShapeDtypeStruct((P, N, D), f32),
            jax.ShapeDtypeStruct((8, D), f32),
        ],
    )(acc, deg, m0)

    hfc, h = pl.pallas_call(
        _tc_final_body,
        grid=(NBLK,),
        in_specs=[
            pl.BlockSpec((P, R, D), lambda i: (0, i, 0)),
            pl.BlockSpec((8, D), lambda i: (0, 0)),
            pl.BlockSpec((P, D), lambda i: (0, 0)),
            pl.BlockSpec((R, 1), lambda i: (i, 0)),
            pl.BlockSpec((R, D), lambda i: (i, 0)),
            pl.BlockSpec((D, D), lambda i: (0, 0)),
            pl.BlockSpec((1, D), lambda i: (0, 0)),
        ],
        out_specs=[
            pl.BlockSpec((R, D), lambda i: (i, 0)),
            pl.BlockSpec((R, D), lambda i: (i, 0)),
        ],
        out_shape=[
            jax.ShapeDtypeStruct((N, D), f32),
            jax.ShapeDtypeStruct((N, D), f32),
        ],
    )(agg, stats, att, m0, temps, fc_w.T, fc_b.reshape(1, D))

    return (hfc, h)


# speedup vs baseline: 1.3219x; 1.0291x over previous
import functools

import jax
import jax.numpy as jnp
from jax import lax
from jax.experimental import pallas as pl
from jax.experimental.pallas import tpu as pltpu
from jax.experimental.pallas import tpu_sc as plsc

N = 10000
E = 320000
P = 3
D = 128

NC = 2
NS = 16
NW = NC * NS
EPT = E // NW
CB = 80
NB_E = EPT // CB
SEG = 25
NSEG = NB_E // SEG
NTRI = (SEG - 4) // 3
CZ = 80
NCHUNK = N // CZ
KMAX_S = (NCHUNK + NS - 1) // NS
KMAX = (NCHUNK + NW - 1) // NW


def _sc_body(src_hbm, dst_hbm, feat_hbm, fidx_hbm, z128_hbm, z16_hbm,
             ones_hbm, acc_out, deg_out, tmp_out,
             acc, dacc, isrc, idst, rows0, rows1, rows2, ones_v, buf16,
             semA, semB, semC, semO, semW, semZ):
    cid = lax.axis_index("c")
    sid = lax.axis_index("s")
    wid = cid * NS + sid

    pltpu.sync_copy(ones_hbm, ones_v)

    def fire(b, rows, sem):
        return pltpu.async_copy(feat_hbm.at[isrc.at[b]], rows, sem)

    def drain(rows, sem):
        pltpu.make_async_copy(feat_hbm.at[isrc.at[0]], rows, sem).wait()

    def scat(b, rows):
        pltpu.sync_copy(rows, acc.at[idst.at[b]], add=True)
        pltpu.async_copy(ones_v, dacc.at[idst.at[b]], semO, add=True)

    def drain_ones():
        for _ in range(SEG):
            pltpu.make_async_copy(ones_v, dacc.at[idst.at[0]], semO).wait()

    def zero_chunks():
        pltpu.sync_copy(z128_hbm, rows0)
        pltpu.sync_copy(z16_hbm, buf16)
        for k in range(KMAX_S):
            chunk = sid + NS * k

            @pl.when(chunk < NCHUNK)
            def _():
                off = pl.multiple_of(chunk * CZ, 8)
                pltpu.async_copy(rows0, acc.at[pl.ds(off, CZ)], semZ)
                pltpu.async_copy(buf16, dacc.at[pl.ds(off, CZ)], semW)

        for k in range(KMAX_S):
            chunk = sid + NS * k

            @pl.when(chunk < NCHUNK)
            def _():
                pltpu.make_async_copy(rows0, acc.at[pl.ds(0, CZ)],
                                      semZ).wait()
                pltpu.make_async_copy(buf16, dacc.at[pl.ds(0, CZ)],
                                      semW).wait()

    zero_chunks()
    plsc.subcore_barrier()

    for p in range(P):
        for seg in range(NSEG):
            pltpu.sync_copy(src_hbm.at[p, wid, pl.ds(seg * SEG, SEG)], isrc)
            pltpu.sync_copy(dst_hbm.at[p, wid, pl.ds(seg * SEG, SEG)], idst)
            fire(0, rows0, semA)
            fire(1, rows1, semB)
            fire(2, rows2, semC)

            def body(j, _):
                b = 3 * j
                drain(rows0, semA)
                scat(b, rows0)
                fire(b + 3, rows0, semA)
                drain(rows1, semB)
                scat(b + 1, rows1)
                fire(b + 4, rows1, semB)
                drain(rows2, semC)
                scat(b + 2, rows2)
                fire(b + 5, rows2, semC)
                return 0

            lax.fori_loop(0, NTRI, body, 0)
            bq = 3 * NTRI
            drain(rows0, semA)
            scat(bq, rows0)
            fire(bq + 3, rows0, semA)
            drain(rows1, semB)
            scat(bq + 1, rows1)
            drain(rows2, semC)
            scat(bq + 2, rows2)
            drain(rows0, semA)
            scat(bq + 3, rows0)
            drain_ones()

        plsc.subcore_barrier()

        for k in range(KMAX_S):
            chunk = sid + NS * k

            @pl.when(chunk < NCHUNK)
            def _():
                off = pl.multiple_of(chunk * CZ, 8)
                pltpu.async_copy(acc.at[pl.ds(off, CZ)],
                                 acc_out.at[p, cid, pl.ds(off, CZ)], semZ)
                pltpu.async_copy(dacc.at[pl.ds(off, CZ)],
                                 deg_out.at[p, cid, pl.ds(off, CZ)], semW)

        for k in range(KMAX_S):
            chunk = sid + NS * k

            @pl.when(chunk < NCHUNK)
            def _():
                pltpu.make_async_copy(acc.at[pl.ds(0, CZ)],
                                      acc_out.at[p, cid, pl.ds(0, CZ)],
                                      semZ).wait()
                pltpu.make_async_copy(dacc.at[pl.ds(0, CZ)],
                                      deg_out.at[p, cid, pl.ds(0, CZ)],
                                      semW).wait()

        if p < P - 1:
            zero_chunks()
        plsc.subcore_barrier()

    for k in range(KMAX):
        chunk = wid + NW * k

        @pl.when(chunk < NCHUNK)
        def _():
            off = pl.multiple_of(chunk * CB, 8)
            pltpu.sync_copy(fidx_hbm.at[pl.ds(off, CB)], isrc.at[0])
            pltpu.async_copy(feat_hbm.at[isrc.at[0]], rows0, semA).wait()
            pltpu.sync_copy(rows0, tmp_out.at[pl.ds(off, CB)])


_sc_call = functools.partial(
    pl.kernel,
    mesh=plsc.VectorSubcoreMesh(core_axis_name="c", subcore_axis_name="s"),
    out_type=[
        jax.ShapeDtypeStruct((P, NC, N, D), jnp.float32),
        jax.ShapeDtypeStruct((P, NC, N, 16), jnp.float32),
        jax.ShapeDtypeStruct((N, D), jnp.float32),
    ],
    scratch_types=[
        pltpu.VMEM_SHARED((N, D), jnp.float32),
        pltpu.VMEM_SHARED((N, 16), jnp.float32),
        pltpu.VMEM((SEG, CB), jnp.int32),
        pltpu.VMEM((SEG, CB), jnp.int32),
        pltpu.VMEM((CB, D), jnp.float32),
        pltpu.VMEM((CB, D), jnp.float32),
        pltpu.VMEM((CB, D), jnp.float32),
        pltpu.VMEM((CB, 16), jnp.float32),
        pltpu.VMEM((CZ, 16), jnp.float32),
        pltpu.SemaphoreType.DMA,
        pltpu.SemaphoreType.DMA,
        pltpu.SemaphoreType.DMA,
        pltpu.SemaphoreType.DMA,
        pltpu.SemaphoreType.DMA,
        pltpu.SemaphoreType.DMA,
    ],
    compiler_params=pltpu.CompilerParams(use_tc_tiling_on_sc=False),
)(_sc_body)


R = 1000
NBLK = N // R


def _tc_reduce_body(acc_ref, deg_ref, m0_ref, agg_ref, stats_ref):
    i = pl.program_id(0)
    a = acc_ref[...]
    d = deg_ref[...]
    deg = d[:, 0, :, 0:1] + d[:, 1, :, 0:1]
    agg = (a[:, 0] + a[:, 1]) / jnp.maximum(deg, 1.0)
    agg_ref[...] = agg
    m = m0_ref[...]
    contrib = jnp.sum(jnp.tanh(agg) * m[None, :, :], axis=1)
    cnt = jnp.sum(m)

    @pl.when(i == 0)
    def _():
        stats_ref[...] = jnp.zeros((8, D), jnp.float32)

    upd = jnp.concatenate(
        [contrib, jnp.full((1, D), cnt, jnp.float32),
         jnp.zeros((4, D), jnp.float32)], axis=0)
    stats_ref[...] = stats_ref[...] + upd


def _tc_final_body(agg_ref, stats_ref, att_ref, m0_ref, tmp_ref,
                   fcwt_ref, fcb_ref, hfc_ref, h_ref):
    stats = stats_ref[...]
    att = att_ref[...]
    sv = jnp.sum(stats[0:P, :] * att, axis=1, keepdims=True)
    cnt = jnp.maximum(stats[P:P + 1, 0:1], 1.0)
    s = sv / cnt
    s = s - jnp.max(s, axis=0, keepdims=True)
    e = jnp.exp(s)
    beta = e / jnp.sum(e, axis=0, keepdims=True)
    agg = agg_ref[...]
    ht = (beta[0:1, 0:1] * agg[0] + beta[1:2, 0:1] * agg[1]
          + beta[2:3, 0:1] * agg[2])
    h = jnp.where(m0_ref[...] > 0.5, ht, tmp_ref[...])
    h_ref[...] = h
    hfc_ref[...] = (jnp.dot(h, fcwt_ref[...],
                            preferred_element_type=jnp.float32)
                    + fcb_ref[...])


def kernel(features, type_mask, adj_matrixes, feature_idxes, fc_w, fc_b, att):
    f32 = jnp.float32
    m0 = (type_mask == 0).astype(f32).reshape(N, 1)
    z128 = jnp.zeros((CZ, D), f32)
    z16 = jnp.zeros((CZ, 16), f32)
    ones16 = jnp.ones((CB, 16), f32)
    src4 = adj_matrixes[:, 0, :].reshape(P, NW, NB_E, CB)
    dst4 = adj_matrixes[:, 1, :].reshape(P, NW, NB_E, CB)

    acc, deg, temps = _sc_call(src4, dst4, features, feature_idxes,
                               z128, z16, ones16)

    agg, stats = pl.pallas_call(
        _tc_reduce_body,
        grid=(NBLK,),
        in_specs=[
            pl.BlockSpec((P, NC, R, D), lambda i: (0, 0, i, 0)),
            pl.BlockSpec((P, NC, R, 16), lambda i: (0, 0, i, 0)),
            pl.BlockSpec((R, 1), lambda i: (i, 0)),
        ],
        out_specs=[
            pl.BlockSpec((P, R, D), lambda i: (0, i, 0)),
            pl.BlockSpec((8, D), lambda i: (0, 0)),
        ],
        out_shape=[
            jax.ShapeDtypeStruct((P, N, D), f32),
            jax.ShapeDtypeStruct((8, D), f32),
        ],
    )(acc, deg, m0)

    hfc, h = pl.pallas_call(
        _tc_final_body,
        grid=(NBLK,),
        in_specs=[
            pl.BlockSpec((P, R, D), lambda i: (0, i, 0)),
            pl.BlockSpec((8, D), lambda i: (0, 0)),
            pl.BlockSpec((P, D), lambda i: (0, 0)),
            pl.BlockSpec((R, 1), lambda i: (i, 0)),
            pl.BlockSpec((R, D), lambda i: (i, 0)),
            pl.BlockSpec((D, D), lambda i: (0, 0)),
            pl.BlockSpec((1, D), lambda i: (0, 0)),
        ],
        out_specs=[
            pl.BlockSpec((R, D), lambda i: (i, 0)),
            pl.BlockSpec((R, D), lambda i: (i, 0)),
        ],
        out_shape=[
            jax.ShapeDtypeStruct((N, D), f32),
            jax.ShapeDtypeStruct((N, D), f32),
        ],
    )(agg, stats, att, m0, temps, fc_w.T, fc_b.reshape(1, D))

    return (hfc, h)
